# baseline (speedup 1.0000x reference)
import jax
import jax.numpy as jnp
from jax import lax
from jax.experimental import pallas as pl
from jax.experimental.pallas import tpu as pltpu

_SB = 32
_NSEM = 8


def kernel(x, pe_weight):
    batch = x.shape[0]
    max_len, d_model = pe_weight.shape
    sb = _SB if batch % _SB == 0 else 1
    n_copies = batch // sb

    def body(pe_ref, out_ref, scratch_ref, sems):
        scratch_ref[...] = jnp.broadcast_to(
            pe_ref[...][None, :, :], scratch_ref.shape
        )
        copies = [
            pltpu.make_async_copy(
                scratch_ref,
                out_ref.at[pl.ds(i * sb, sb)],
                sems.at[i % _NSEM],
            )
            for i in range(n_copies)
        ]
        for c in copies:
            c.start()
        for c in copies:
            c.wait()

    return pl.pallas_call(
        body,
        in_specs=[pl.BlockSpec(memory_space=pltpu.MemorySpace.VMEM)],
        out_specs=pl.BlockSpec(memory_space=pl.ANY),
        out_shape=jax.ShapeDtypeStruct((batch, max_len, d_model), pe_weight.dtype),
        scratch_shapes=[
            pltpu.VMEM((sb, max_len, d_model), pe_weight.dtype),
            pltpu.SemaphoreType.DMA((_NSEM,)),
        ],
    )(pe_weight)


# speedup vs baseline: 1.0407x; 1.0407x over previous
import jax
import jax.numpy as jnp
from jax.experimental import pallas as pl
from jax.experimental.pallas import tpu as pltpu

_BB = 32


def _bcast_body(pe_ref, out_ref):
    out_ref[...] = jnp.broadcast_to(pe_ref[...][None, :, :], out_ref.shape)


def kernel(x, pe_weight):
    batch = x.shape[0]
    max_len, d_model = pe_weight.shape
    bb = _BB if batch % _BB == 0 else 1
    return pl.pallas_call(
        _bcast_body,
        grid=(batch // bb,),
        in_specs=[pl.BlockSpec(memory_space=pltpu.MemorySpace.VMEM)],
        out_specs=pl.BlockSpec((bb, max_len, d_model), lambda i: (i, 0, 0)),
        out_shape=jax.ShapeDtypeStruct((batch, max_len, d_model), pe_weight.dtype),
    )(pe_weight)
